# baseline (speedup 1.0000x reference)
import jax
import jax.numpy as jnp
from jax import lax
from jax.experimental import pallas as pl
from jax.experimental.pallas import tpu as pltpu
from jax.experimental.pallas import tpu_sc as plsc

N = 10000
E = 320000
D = 128

NC = 2
NT = 16
NW = NC * NT
CW = 128
CPT = 80
EP = NW * CPT * CW
NP = 10240
RPT = NP // NT
BLK = 512

_MESH = plsc.VectorSubcoreMesh(core_axis_name="c", subcore_axis_name="s")


def _segsum_body(tab, srcr, dstr, zrows, acc_o, idx_s, idx_d, rows, acc_sh, sem):
    cc = lax.axis_index("c")
    ss = lax.axis_index("s")
    wid = ss * NC + cc
    pltpu.sync_copy(zrows, acc_sh.at[pl.ds(ss * RPT, RPT)])
    pltpu.sync_copy(srcr.at[pl.ds(wid * CPT, CPT)], idx_s)
    pltpu.sync_copy(dstr.at[pl.ds(wid * CPT, CPT)], idx_d)
    plsc.subcore_barrier()

    def chunk(j, carry):
        pltpu.async_copy(tab.at[idx_s.at[j]], rows, sem).wait()
        pltpu.sync_copy(rows, acc_sh.at[idx_d.at[j]], add=True)
        return carry

    lax.fori_loop(0, CPT, chunk, 0)
    plsc.subcore_barrier()
    base = cc * NP + ss * RPT
    pltpu.sync_copy(acc_sh.at[pl.ds(ss * RPT, RPT)], acc_o.at[pl.ds(base, RPT)])


def _make_segsum():
    return pl.kernel(
        _segsum_body,
        out_type=jax.ShapeDtypeStruct((2 * NP, D), jnp.float32),
        mesh=_MESH,
        scratch_types=[
            pltpu.VMEM((CPT, CW), jnp.int32),
            pltpu.VMEM((CPT, CW), jnp.int32),
            pltpu.VMEM((CW, D), jnp.float32),
            pltpu.VMEM_SHARED((NP, D), jnp.float32),
            pltpu.SemaphoreType.DMA,
        ],
    )


def _mm_t(a, w):
    return lax.dot_general(a, w, (((1,), (1,)), ((), ())),
                           preferred_element_type=jnp.float32)


def _mean(a0, a1, c0, c1):
    cnt = jnp.maximum(c0[...][:, :1] + c1[...][:, :1], 1.0)
    return (a0[...] + a1[...]) / cnt


def _layer1_body(x_ref, a0, a1, c0, c1, wl, bl, wr, h_ref):
    mean = _mean(a0, a1, c0, c1)
    h_ref[...] = jnp.maximum(
        _mm_t(mean, wl[...]) + bl[...] + _mm_t(x_ref[...], wr[...]), 0.0)


def _layer2_body(h_ref, a0, a1, c0, c1, wl, bl, wr, wlin, blin, o_ref):
    mean = _mean(a0, a1, c0, c1)
    h2 = jnp.maximum(
        _mm_t(mean, wl[...]) + bl[...] + _mm_t(h_ref[...], wr[...]), 0.0)
    o_ref[...] = _mm_t(h2, wlin[...]) + blin[0, 0]


_NPB = NP // BLK


def _row_spec(i):
    return (i, 0)


def _row_spec_hi(i):
    return (i + _NPB, 0)


def _fixed(i):
    return (0, 0)


def _layer1_call(xp, aggs, cnts, Wl1, bl1, Wr1):
    return pl.pallas_call(
        _layer1_body,
        grid=(_NPB,),
        in_specs=[
            pl.BlockSpec((BLK, D), _row_spec),
            pl.BlockSpec((BLK, D), _row_spec),
            pl.BlockSpec((BLK, D), _row_spec_hi),
            pl.BlockSpec((BLK, D), _row_spec),
            pl.BlockSpec((BLK, D), _row_spec_hi),
            pl.BlockSpec((D, D), _fixed),
            pl.BlockSpec((1, D), _fixed),
            pl.BlockSpec((D, D), _fixed),
        ],
        out_specs=pl.BlockSpec((BLK, D), _row_spec),
        out_shape=jax.ShapeDtypeStruct((NP, D), jnp.float32),
    )(xp, aggs, aggs, cnts, cnts, Wl1, bl1.reshape(1, D), Wr1)


def _layer2_call(h, aggs, cnts, Wl2, bl2, Wr2, Wlin, blin):
    return pl.pallas_call(
        _layer2_body,
        grid=(_NPB,),
        in_specs=[
            pl.BlockSpec((BLK, D), _row_spec),
            pl.BlockSpec((BLK, D), _row_spec),
            pl.BlockSpec((BLK, D), _row_spec_hi),
            pl.BlockSpec((BLK, D), _row_spec),
            pl.BlockSpec((BLK, D), _row_spec_hi),
            pl.BlockSpec((D, D), _fixed),
            pl.BlockSpec((1, D), _fixed),
            pl.BlockSpec((D, D), _fixed),
            pl.BlockSpec((8, D), _fixed),
            pl.BlockSpec((1, 1), _fixed),
        ],
        out_specs=pl.BlockSpec((BLK, 8), _row_spec),
        out_shape=jax.ShapeDtypeStruct((NP, 8), jnp.float32),
    )(h, aggs, aggs, cnts, cnts, Wl2, bl2.reshape(1, D), Wr2,
      jnp.zeros((8, D), jnp.float32).at[0].set(Wlin[0]), blin.reshape(1, 1))


def kernel(x, edge_index, Wl1, bl1, Wr1, Wl2, bl2, Wr2, Wlin, blin):
    src = edge_index[0]
    dst = edge_index[1]
    pad = EP - E
    src_p = jnp.concatenate([src, jnp.zeros((pad,), jnp.int32)]).reshape(EP // CW, CW)
    dst_p = jnp.concatenate([dst, jnp.full((pad,), N, jnp.int32)]).reshape(EP // CW, CW)
    xp = jnp.zeros((NP, D), jnp.float32).at[:N].set(x)

    zrows = jnp.zeros((RPT, D), jnp.float32)

    cnts = _make_segsum()(jnp.ones((8, D), jnp.float32), src_p * 0, dst_p, zrows)
    aggs1 = _make_segsum()(x, src_p, dst_p, zrows)
    h = _layer1_call(xp, aggs1, cnts, Wl1, bl1, Wr1)
    aggs2 = _make_segsum()(h, src_p, dst_p, zrows)
    out = _layer2_call(h, aggs2, cnts, Wl2, bl2, Wr2, Wlin, blin)
    return out[:N, 0]
